# baseline (speedup 1.0000x reference)
import functools

import jax
import jax.numpy as jnp
from jax import lax
from jax.experimental import pallas as pl
from jax.experimental.pallas import tpu as pltpu
from jax.experimental.pallas import tpu_sc as plsc

N = 100000
E = 3200000
NPAD = 100096
ROWS = 782
SLICE = NPAD // 16
CHUNK = 2000
CHUNK_A = 25000
NC = 2
NS = 16
NW = NC * NS
EPW = E // NW
NT_AB = EPW // CHUNK
NT_A = EPW // CHUNK_A

_mesh = plsc.VectorSubcoreMesh(
    core_axis_name="c", subcore_axis_name="s", num_cores=NC, num_subcores=NS)
_sc_params = pltpu.CompilerParams(needs_layout_passes=False)

_f32 = jnp.float32
_i32 = jnp.int32
_out2 = jax.ShapeDtypeStruct((NC * NPAD,), _f32)
_out4 = jax.ShapeDtypeStruct((2 * NC * NPAD,), _f32)


def _fill_zero(bounce_v):
    z16 = jnp.zeros((16,), _f32)

    def zb(i, _):
        bounce_v[pl.ds(i * 16, 16)] = z16
        return _

    lax.fori_loop(0, SLICE // 16, zb, None)


def _zero_acc(bounce_v, acc_sh, sid):
    _fill_zero(bounce_v)
    pltpu.sync_copy(bounce_v, acc_sh.at[pl.ds(sid * SLICE, SLICE)])


def _write_out(acc_sh, bounce_v, out_hbm, obase, sid):
    pltpu.sync_copy(acc_sh.at[pl.ds(sid * SLICE, SLICE)], bounce_v)
    pltpu.sync_copy(bounce_v, out_hbm.at[pl.ds(obase + sid * SLICE, SLICE)])


@functools.partial(
    pl.kernel,
    out_type=_out2,
    mesh=_mesh,
    compiler_params=_sc_params,
    scratch_types=[
        pltpu.VMEM_SHARED((NPAD,), _f32),
        pltpu.VMEM((SLICE,), _f32),
        pltpu.VMEM((CHUNK_A,), _i32),
        pltpu.VMEM((CHUNK_A,), _i32),
        pltpu.VMEM((CHUNK_A,), _f32),
        pltpu.VMEM((CHUNK_A,), _f32),
        pltpu.SemaphoreType.DMA,
        pltpu.SemaphoreType.DMA,
        pltpu.SemaphoreType.DMA,
        pltpu.SemaphoreType.DMA,
    ],
)
def _sc_degree(col_hbm, w_hbm, out_hbm, acc_sh, bounce_v, col_v0, col_v1,
               w_v0, w_v1, sem0, sem1, ssem0, ssem1):
    cid = lax.axis_index("c")
    sid = lax.axis_index("s")
    wid = sid * NC + cid
    base = wid * EPW
    sems = (sem0, sem1)
    ssems = (ssem0, ssem1)
    col_b = (col_v0, col_v1)
    w_b = (w_v0, w_v1)

    def start(t, b):
        sl = pl.ds(base + t * CHUNK_A, CHUNK_A)
        pltpu.async_copy(col_hbm.at[sl], col_b[b], sems[b])
        pltpu.async_copy(w_hbm.at[sl], w_b[b], sems[b])

    def wait(t, b):
        sl = pl.ds(base + t * CHUNK_A, CHUNK_A)
        pltpu.make_async_copy(col_hbm.at[sl], col_b[b], sems[b]).wait()
        pltpu.make_async_copy(w_hbm.at[sl], w_b[b], sems[b]).wait()

    def scat_wait(b):
        pltpu.make_async_copy(
            w_b[b], acc_sh.at[col_b[b]], ssems[b]).wait()

    _zero_acc(bounce_v, acc_sh, sid)
    plsc.subcore_barrier()

    start(0, 0)

    def body(i, _):
        for b in (0, 1):
            t = 2 * i + b

            def free_and_prefetch():
                start(t + 1, 1 - b)

            if b == 0:
                pl.when(i > 0)(lambda: scat_wait(1))
                free_and_prefetch()
            else:
                scat_wait(0)
                pl.when(i < NT_A // 2 - 1)(free_and_prefetch)
            wait(t, b)
            pltpu.async_copy(w_b[b], acc_sh.at[col_b[b]], ssems[b], add=True)
        return _

    lax.fori_loop(0, NT_A // 2, body, None)
    scat_wait(1)
    plsc.subcore_barrier()
    _write_out(acc_sh, bounce_v, out_hbm, cid * NPAD, sid)


@functools.partial(
    pl.kernel,
    out_type=_out2,
    mesh=_mesh,
    compiler_params=_sc_params,
    scratch_types=[
        pltpu.VMEM_SHARED((NPAD,), _f32),
        pltpu.VMEM((SLICE,), _f32),
        pltpu.VMEM((NPAD,), _f32),
        pltpu.VMEM((CHUNK,), _i32),
        pltpu.VMEM((CHUNK,), _i32),
        pltpu.VMEM((CHUNK,), _i32),
        pltpu.VMEM((CHUNK,), _i32),
        pltpu.VMEM((CHUNK,), _f32),
        pltpu.VMEM((CHUNK,), _f32),
        pltpu.VMEM((CHUNK,), _f32),
        pltpu.VMEM((CHUNK,), _f32),
        pltpu.SemaphoreType.DMA,
        pltpu.SemaphoreType.DMA,
        pltpu.SemaphoreType.DMA,
        pltpu.SemaphoreType.DMA,
    ],
)
def _sc_spmv1(row_hbm, col_hbm, w_hbm, du_hbm, out_hbm,
              acc_sh, bounce_v, tab_v, row_v0, row_v1, col_v0, col_v1,
              w_v0, w_v1, m_v0, m_v1, sem0, sem1, ssem0, ssem1):
    cid = lax.axis_index("c")
    sid = lax.axis_index("s")
    wid = sid * NC + cid
    base = wid * EPW
    sems = (sem0, sem1)
    ssems = (ssem0, ssem1)
    row_b = (row_v0, row_v1)
    col_b = (col_v0, col_v1)
    w_b = (w_v0, w_v1)
    m_b = (m_v0, m_v1)

    def start(t, b):
        sl = pl.ds(base + t * CHUNK, CHUNK)
        pltpu.async_copy(row_hbm.at[sl], row_b[b], sems[b])
        pltpu.async_copy(col_hbm.at[sl], col_b[b], sems[b])
        pltpu.async_copy(w_hbm.at[sl], w_b[b], sems[b])

    def wait(t, b):
        sl = pl.ds(base + t * CHUNK, CHUNK)
        pltpu.make_async_copy(row_hbm.at[sl], row_b[b], sems[b]).wait()
        pltpu.make_async_copy(col_hbm.at[sl], col_b[b], sems[b]).wait()
        pltpu.make_async_copy(w_hbm.at[sl], w_b[b], sems[b]).wait()

    def scat_wait(b):
        pltpu.make_async_copy(
            m_b[b], acc_sh.at[col_b[b]], ssems[b]).wait()

    def gather_mul(b):
        rv, wv, mv = row_b[b], w_b[b], m_b[b]

        @plsc.parallel_loop(0, CHUNK, 16, unroll=16)
        def gbody(i):
            sl = pl.ds(i, 16)
            t16 = plsc.load_gather(tab_v, [rv[sl]])
            mv[sl] = wv[sl] * t16

    start(0, 0)
    _zero_acc(bounce_v, acc_sh, sid)
    pltpu.sync_copy(du_hbm.at[pl.ds(NPAD, NPAD)], tab_v)
    plsc.subcore_barrier()

    def body(i, _):
        for b in (0, 1):
            t = 2 * i + b

            def free_and_prefetch():
                start(t + 1, 1 - b)

            if b == 0:
                pl.when(i > 0)(lambda: scat_wait(1))
                free_and_prefetch()
            else:
                scat_wait(0)
                pl.when(i < NT_AB // 2 - 1)(free_and_prefetch)
            wait(t, b)
            gather_mul(b)
            pltpu.async_copy(m_b[b], acc_sh.at[col_b[b]], ssems[b], add=True)
        return _

    lax.fori_loop(0, NT_AB // 2, body, None)
    scat_wait(1)
    plsc.subcore_barrier()
    _write_out(acc_sh, bounce_v, out_hbm, cid * NPAD, sid)


@functools.partial(
    pl.kernel,
    out_type=_out4,
    mesh=_mesh,
    compiler_params=_sc_params,
    scratch_types=[
        pltpu.VMEM_SHARED((NPAD,), _f32),
        pltpu.VMEM_SHARED((NPAD,), _f32),
        pltpu.VMEM((NPAD,), _i32),
        pltpu.VMEM((CHUNK,), _i32),
        pltpu.VMEM((CHUNK,), _i32),
        pltpu.VMEM((CHUNK,), _i32),
        pltpu.VMEM((CHUNK,), _i32),
        pltpu.VMEM((CHUNK,), _f32),
        pltpu.VMEM((CHUNK,), _f32),
        pltpu.VMEM((CHUNK,), _f32),
        pltpu.VMEM((CHUNK,), _f32),
        pltpu.SemaphoreType.DMA,
        pltpu.SemaphoreType.DMA,
    ],
)
def _sc_spmv2(row_hbm, col_hbm, w_hbm, vpk_hbm, out_hbm,
              acc0_sh, acc1_sh, tab_v, row_v0, row_v1,
              col_v0, col_v1, w_v0, w_v1, m0_v, m1_v,
              sem0, sem1):
    cid = lax.axis_index("c")
    sid = lax.axis_index("s")
    wid = sid * NC + cid
    base = wid * EPW
    sems = (sem0, sem1)
    row_b = (row_v0, row_v1)
    col_b = (col_v0, col_v1)
    w_b = (w_v0, w_v1)

    def start(t, b):
        sl = pl.ds(base + t * CHUNK, CHUNK)
        pltpu.async_copy(row_hbm.at[sl], row_b[b], sems[b])
        pltpu.async_copy(col_hbm.at[sl], col_b[b], sems[b])
        pltpu.async_copy(w_hbm.at[sl], w_b[b], sems[b])

    def wait(t, b):
        sl = pl.ds(base + t * CHUNK, CHUNK)
        pltpu.make_async_copy(row_hbm.at[sl], row_b[b], sems[b]).wait()
        pltpu.make_async_copy(col_hbm.at[sl], col_b[b], sems[b]).wait()
        pltpu.make_async_copy(w_hbm.at[sl], w_b[b], sems[b]).wait()

    hi_mask = jnp.full((16,), -65536, _i32)

    def gather_mul(b):
        rv, wv, m0v, m1v = row_b[b], w_b[b], m0_v, m1_v

        @plsc.parallel_loop(0, CHUNK, 16, unroll=16)
        def gbody(i):
            sl = pl.ds(i, 16)
            pk = plsc.load_gather(tab_v, [rv[sl]])
            v0 = plsc.bitcast(pk & hi_mask, _f32)
            v1 = plsc.bitcast(pk << 16, _f32)
            w16 = wv[sl]
            m0v[sl] = w16 * v0
            m1v[sl] = w16 * v1

    start(0, 0)
    z16 = jnp.zeros((16,), _f32)

    def zb(i, _):
        m0_v[pl.ds(i * 16, 16)] = z16
        return _

    lax.fori_loop(0, CHUNK // 16, zb, None)
    zbase = sid * SLICE
    for acc in (acc0_sh, acc1_sh):
        for k in range(3):
            pltpu.sync_copy(m0_v, acc.at[pl.ds(zbase + k * 2000, 2000)])
        pltpu.sync_copy(m0_v.at[pl.ds(0, 256)],
                        acc.at[pl.ds(zbase + 6000, 256)])
    pltpu.sync_copy(vpk_hbm, tab_v)
    plsc.subcore_barrier()

    def body(i, _):
        for b in (0, 1):
            t = 2 * i + b

            def prefetch():
                start(t + 1, 1 - b)

            if b == 0:
                prefetch()
            else:
                pl.when(i < NT_AB // 2 - 1)(prefetch)
            wait(t, b)
            gather_mul(b)
            pltpu.sync_copy(m0_v, acc0_sh.at[col_b[b]], add=True)
            pltpu.sync_copy(m1_v, acc1_sh.at[col_b[b]], add=True)
        return _

    lax.fori_loop(0, NT_AB // 2, body, None)
    plsc.subcore_barrier()
    for f, acc in ((0, acc0_sh), (1, acc1_sh)):
        obase = cid * 2 * NPAD + f * NPAD + sid * SLICE
        for k in range(3):
            pltpu.sync_copy(acc.at[pl.ds(zbase + k * 2000, 2000)], m0_v)
            pltpu.sync_copy(m0_v, out_hbm.at[pl.ds(obase + k * 2000, 2000)])
        pltpu.sync_copy(acc.at[pl.ds(zbase + 6000, 256)],
                        m0_v.at[pl.ds(0, 256)])
        pltpu.sync_copy(m0_v.at[pl.ds(0, 256)],
                        out_hbm.at[pl.ds(obase + 6000, 256)])


def _tc1_body(deg, xr, du):
    d0 = deg[0:ROWS, :]
    d1 = deg[ROWS:2 * ROWS, :]
    dd = lax.rsqrt(d0 + d1 + 1.0)
    du[0:ROWS, :] = dd
    du[ROWS:2 * ROWS, :] = dd * xr[...]


def _tc2_body(accb, du, w1, b1, w2, v_out, vpk_out):
    a0 = accb[0:ROWS, :]
    a1 = accb[ROWS:2 * ROWS, :]
    d = du[0:ROWS, :]
    u = du[ROWS:2 * ROWS, :]
    y = d * (a0 + a1 + u)
    z0 = jnp.zeros_like(y)
    z1 = jnp.zeros_like(y)
    for j in range(16):
        h = jnp.maximum(y * w1[0, j] + b1[j], 0.0)
        z0 = z0 + h * w2[j, 0]
        z1 = z1 + h * w2[j, 1]
    v0 = d * z0
    v1 = d * z1
    v_out[0:ROWS, :] = v0
    v_out[ROWS:2 * ROWS, :] = v1
    hi = lax.bitcast_convert_type(
        v0.astype(jnp.bfloat16), jnp.uint16).astype(jnp.uint32) << 16
    lo = lax.bitcast_convert_type(
        v1.astype(jnp.bfloat16), jnp.uint16).astype(jnp.uint32)
    vpk_out[...] = lax.bitcast_convert_type(hi | lo, _i32)


def _tc3_body(accc, v, du, b2, o0, o1):
    a00 = accc[0:ROWS, :]
    a10 = accc[ROWS:2 * ROWS, :]
    a01 = accc[2 * ROWS:3 * ROWS, :]
    a11 = accc[3 * ROWS:4 * ROWS, :]
    v0 = v[0:ROWS, :]
    v1 = v[ROWS:2 * ROWS, :]
    d = du[0:ROWS, :]
    q0 = d * (a00 + a01 + v0) + b2[0]
    q1 = d * (a10 + a11 + v1) + b2[1]
    m = jnp.maximum(q0, q1)
    lz = m + jnp.log(jnp.exp(q0 - m) + jnp.exp(q1 - m))
    o0[...] = q0 - lz
    o1[...] = q1 - lz


_vspec = pl.BlockSpec(memory_space=pltpu.VMEM)
_smem_spec = pl.BlockSpec(memory_space=pltpu.SMEM)


def _sds(rows, dtype=_f32):
    return jax.ShapeDtypeStruct((rows, 128), dtype)


def kernel(x, edge_index, edge_weight, W1, b1, W2, b2):
    rowe = edge_index[0]
    cole = edge_index[1]
    we = edge_weight
    xp = jnp.pad(x[:, 0], (0, NPAD - N)).reshape(ROWS, 128)

    deg = _sc_degree(cole, we)

    du = pl.pallas_call(
        _tc1_body,
        in_specs=[_vspec, _vspec],
        out_specs=_vspec,
        out_shape=_sds(2 * ROWS),
    )(deg.reshape(2 * ROWS, 128), xp)

    accB = _sc_spmv1(rowe, cole, we, du.reshape(NC * NPAD))

    v, vpk = pl.pallas_call(
        _tc2_body,
        in_specs=[_vspec, _vspec, _smem_spec, _smem_spec, _smem_spec],
        out_specs=[_vspec, _vspec],
        out_shape=[_sds(2 * ROWS), _sds(ROWS, _i32)],
    )(accB.reshape(2 * ROWS, 128), du, W1, b1, W2)

    accC = _sc_spmv2(rowe, cole, we, vpk.reshape(NPAD))

    o0, o1 = pl.pallas_call(
        _tc3_body,
        in_specs=[_vspec, _vspec, _vspec, _smem_spec],
        out_specs=[_vspec, _vspec],
        out_shape=[_sds(ROWS), _sds(ROWS)],
    )(accC.reshape(4 * ROWS, 128), v, du, b2)

    return jnp.stack([o0.reshape(NPAD)[:N], o1.reshape(NPAD)[:N]], axis=1)


# speedup vs baseline: 1.0270x; 1.0270x over previous
import functools

import jax
import jax.numpy as jnp
from jax import lax
from jax.experimental import pallas as pl
from jax.experimental.pallas import tpu as pltpu
from jax.experimental.pallas import tpu_sc as plsc

N = 100000
E = 3200000
NPAD = 100096
ROWS = 782
SLICE = NPAD // 16
CHUNK = 2000
CHUNK_A = 25000
NC = 2
NS = 16
NW = NC * NS
EPW = E // NW
NT_AB = EPW // CHUNK
NT_A = EPW // CHUNK_A

_mesh = plsc.VectorSubcoreMesh(
    core_axis_name="c", subcore_axis_name="s", num_cores=NC, num_subcores=NS)
_sc_params = pltpu.CompilerParams(needs_layout_passes=False)

_f32 = jnp.float32
_i32 = jnp.int32
_out2 = jax.ShapeDtypeStruct((NC * NPAD,), _f32)
_out4 = jax.ShapeDtypeStruct((2 * NC * NPAD,), _f32)


def _fill_zero(bounce_v):
    z16 = jnp.zeros((16,), _f32)

    def zb(i, _):
        bounce_v[pl.ds(i * 16, 16)] = z16
        return _

    lax.fori_loop(0, SLICE // 16, zb, None)


def _zero_acc(bounce_v, acc_sh, sid):
    _fill_zero(bounce_v)
    pltpu.sync_copy(bounce_v, acc_sh.at[pl.ds(sid * SLICE, SLICE)])


def _write_out(acc_sh, bounce_v, out_hbm, obase, sid):
    pltpu.sync_copy(acc_sh.at[pl.ds(sid * SLICE, SLICE)], bounce_v)
    pltpu.sync_copy(bounce_v, out_hbm.at[pl.ds(obase + sid * SLICE, SLICE)])


@functools.partial(
    pl.kernel,
    out_type=_out2,
    mesh=_mesh,
    compiler_params=_sc_params,
    scratch_types=[
        pltpu.VMEM_SHARED((NPAD,), _f32),
        pltpu.VMEM((SLICE,), _f32),
        pltpu.VMEM((CHUNK_A,), _i32),
        pltpu.VMEM((CHUNK_A,), _i32),
        pltpu.VMEM((CHUNK_A,), _f32),
        pltpu.VMEM((CHUNK_A,), _f32),
        pltpu.SemaphoreType.DMA,
        pltpu.SemaphoreType.DMA,
        pltpu.SemaphoreType.DMA,
        pltpu.SemaphoreType.DMA,
    ],
)
def _sc_degree(col_hbm, w_hbm, out_hbm, acc_sh, bounce_v, col_v0, col_v1,
               w_v0, w_v1, sem0, sem1, ssem0, ssem1):
    cid = lax.axis_index("c")
    sid = lax.axis_index("s")
    wid = sid * NC + cid
    base = wid * EPW
    sems = (sem0, sem1)
    ssems = (ssem0, ssem1)
    col_b = (col_v0, col_v1)
    w_b = (w_v0, w_v1)

    def start(t, b):
        sl = pl.ds(base + t * CHUNK_A, CHUNK_A)
        pltpu.async_copy(col_hbm.at[sl], col_b[b], sems[b])
        pltpu.async_copy(w_hbm.at[sl], w_b[b], sems[b])

    def wait(t, b):
        sl = pl.ds(base + t * CHUNK_A, CHUNK_A)
        pltpu.make_async_copy(col_hbm.at[sl], col_b[b], sems[b]).wait()
        pltpu.make_async_copy(w_hbm.at[sl], w_b[b], sems[b]).wait()

    def scat_wait(b):
        pltpu.make_async_copy(
            w_b[b], acc_sh.at[col_b[b]], ssems[b]).wait()

    _zero_acc(bounce_v, acc_sh, sid)
    plsc.subcore_barrier()

    start(0, 0)

    def body(i, _):
        for b in (0, 1):
            t = 2 * i + b

            def free_and_prefetch():
                start(t + 1, 1 - b)

            if b == 0:
                pl.when(i > 0)(lambda: scat_wait(1))
                free_and_prefetch()
            else:
                scat_wait(0)
                pl.when(i < NT_A // 2 - 1)(free_and_prefetch)
            wait(t, b)
            pltpu.async_copy(w_b[b], acc_sh.at[col_b[b]], ssems[b], add=True)
        return _

    lax.fori_loop(0, NT_A // 2, body, None)
    scat_wait(1)
    plsc.subcore_barrier()
    _write_out(acc_sh, bounce_v, out_hbm, cid * NPAD, sid)


@functools.partial(
    pl.kernel,
    out_type=_out2,
    mesh=_mesh,
    compiler_params=_sc_params,
    scratch_types=[
        pltpu.VMEM_SHARED((NPAD,), _f32),
        pltpu.VMEM((SLICE,), _f32),
        pltpu.VMEM((NPAD,), _f32),
        pltpu.VMEM((CHUNK,), _i32),
        pltpu.VMEM((CHUNK,), _i32),
        pltpu.VMEM((CHUNK,), _i32),
        pltpu.VMEM((CHUNK,), _i32),
        pltpu.VMEM((CHUNK,), _f32),
        pltpu.VMEM((CHUNK,), _f32),
        pltpu.VMEM((CHUNK,), _f32),
        pltpu.VMEM((CHUNK,), _f32),
        pltpu.SemaphoreType.DMA,
        pltpu.SemaphoreType.DMA,
        pltpu.SemaphoreType.DMA,
        pltpu.SemaphoreType.DMA,
    ],
)
def _sc_spmv1(row_hbm, col_hbm, w_hbm, du_hbm, out_hbm,
              acc_sh, bounce_v, tab_v, row_v0, row_v1, col_v0, col_v1,
              w_v0, w_v1, m_v0, m_v1, sem0, sem1, ssem0, ssem1):
    cid = lax.axis_index("c")
    sid = lax.axis_index("s")
    wid = sid * NC + cid
    base = wid * EPW
    sems = (sem0, sem1)
    ssems = (ssem0, ssem1)
    row_b = (row_v0, row_v1)
    col_b = (col_v0, col_v1)
    w_b = (w_v0, w_v1)
    m_b = (m_v0, m_v1)

    def start(t, b):
        sl = pl.ds(base + t * CHUNK, CHUNK)
        pltpu.async_copy(row_hbm.at[sl], row_b[b], sems[b])
        pltpu.async_copy(col_hbm.at[sl], col_b[b], sems[b])
        pltpu.async_copy(w_hbm.at[sl], w_b[b], sems[b])

    def wait(t, b):
        sl = pl.ds(base + t * CHUNK, CHUNK)
        pltpu.make_async_copy(row_hbm.at[sl], row_b[b], sems[b]).wait()
        pltpu.make_async_copy(col_hbm.at[sl], col_b[b], sems[b]).wait()
        pltpu.make_async_copy(w_hbm.at[sl], w_b[b], sems[b]).wait()

    def scat_wait(b):
        pltpu.make_async_copy(
            m_b[b], acc_sh.at[col_b[b]], ssems[b]).wait()

    def gather_mul(b):
        rv, wv, mv = row_b[b], w_b[b], m_b[b]

        @plsc.parallel_loop(0, CHUNK, 16, unroll=8)
        def gbody(i):
            sl = pl.ds(i, 16)
            t16 = plsc.load_gather(tab_v, [rv[sl]])
            mv[sl] = wv[sl] * t16

    start(0, 0)
    _zero_acc(bounce_v, acc_sh, sid)
    pltpu.sync_copy(du_hbm.at[pl.ds(NPAD, NPAD)], tab_v)
    plsc.subcore_barrier()

    def body(i, _):
        for b in (0, 1):
            t = 2 * i + b

            def free_and_prefetch():
                start(t + 1, 1 - b)

            if b == 0:
                pl.when(i > 0)(lambda: scat_wait(1))
                free_and_prefetch()
            else:
                scat_wait(0)
                pl.when(i < NT_AB // 2 - 1)(free_and_prefetch)
            wait(t, b)
            gather_mul(b)
            pltpu.async_copy(m_b[b], acc_sh.at[col_b[b]], ssems[b], add=True)
        return _

    lax.fori_loop(0, NT_AB // 2, body, None)
    scat_wait(1)
    plsc.subcore_barrier()
    _write_out(acc_sh, bounce_v, out_hbm, cid * NPAD, sid)


@functools.partial(
    pl.kernel,
    out_type=_out4,
    mesh=_mesh,
    compiler_params=_sc_params,
    scratch_types=[
        pltpu.VMEM_SHARED((NPAD,), _f32),
        pltpu.VMEM_SHARED((NPAD,), _f32),
        pltpu.VMEM((NPAD,), _i32),
        pltpu.VMEM((CHUNK,), _i32),
        pltpu.VMEM((CHUNK,), _i32),
        pltpu.VMEM((CHUNK,), _i32),
        pltpu.VMEM((CHUNK,), _i32),
        pltpu.VMEM((CHUNK,), _f32),
        pltpu.VMEM((CHUNK,), _f32),
        pltpu.VMEM((CHUNK,), _f32),
        pltpu.VMEM((CHUNK,), _f32),
        pltpu.SemaphoreType.DMA,
        pltpu.SemaphoreType.DMA,
    ],
)
def _sc_spmv2(row_hbm, col_hbm, w_hbm, vpk_hbm, out_hbm,
              acc0_sh, acc1_sh, tab_v, row_v0, row_v1,
              col_v0, col_v1, w_v0, w_v1, m0_v, m1_v,
              sem0, sem1):
    cid = lax.axis_index("c")
    sid = lax.axis_index("s")
    wid = sid * NC + cid
    base = wid * EPW
    sems = (sem0, sem1)
    row_b = (row_v0, row_v1)
    col_b = (col_v0, col_v1)
    w_b = (w_v0, w_v1)

    def start(t, b):
        sl = pl.ds(base + t * CHUNK, CHUNK)
        pltpu.async_copy(row_hbm.at[sl], row_b[b], sems[b])
        pltpu.async_copy(col_hbm.at[sl], col_b[b], sems[b])
        pltpu.async_copy(w_hbm.at[sl], w_b[b], sems[b])

    def wait(t, b):
        sl = pl.ds(base + t * CHUNK, CHUNK)
        pltpu.make_async_copy(row_hbm.at[sl], row_b[b], sems[b]).wait()
        pltpu.make_async_copy(col_hbm.at[sl], col_b[b], sems[b]).wait()
        pltpu.make_async_copy(w_hbm.at[sl], w_b[b], sems[b]).wait()

    hi_mask = jnp.full((16,), -65536, _i32)

    def gather_mul(b):
        rv, wv, m0v, m1v = row_b[b], w_b[b], m0_v, m1_v

        @plsc.parallel_loop(0, CHUNK, 16, unroll=8)
        def gbody(i):
            sl = pl.ds(i, 16)
            pk = plsc.load_gather(tab_v, [rv[sl]])
            v0 = plsc.bitcast(pk & hi_mask, _f32)
            v1 = plsc.bitcast(pk << 16, _f32)
            w16 = wv[sl]
            m0v[sl] = w16 * v0
            m1v[sl] = w16 * v1

    start(0, 0)
    z16 = jnp.zeros((16,), _f32)

    def zb(i, _):
        m0_v[pl.ds(i * 16, 16)] = z16
        return _

    lax.fori_loop(0, CHUNK // 16, zb, None)
    zbase = sid * SLICE
    for acc in (acc0_sh, acc1_sh):
        for k in range(3):
            pltpu.sync_copy(m0_v, acc.at[pl.ds(zbase + k * 2000, 2000)])
        pltpu.sync_copy(m0_v.at[pl.ds(0, 256)],
                        acc.at[pl.ds(zbase + 6000, 256)])
    pltpu.sync_copy(vpk_hbm, tab_v)
    plsc.subcore_barrier()

    def body(i, _):
        for b in (0, 1):
            t = 2 * i + b

            def prefetch():
                start(t + 1, 1 - b)

            if b == 0:
                prefetch()
            else:
                pl.when(i < NT_AB // 2 - 1)(prefetch)
            wait(t, b)
            gather_mul(b)
            pltpu.sync_copy(m0_v, acc0_sh.at[col_b[b]], add=True)
            pltpu.sync_copy(m1_v, acc1_sh.at[col_b[b]], add=True)
        return _

    lax.fori_loop(0, NT_AB // 2, body, None)
    plsc.subcore_barrier()
    for f, acc in ((0, acc0_sh), (1, acc1_sh)):
        obase = cid * 2 * NPAD + f * NPAD + sid * SLICE
        for k in range(3):
            pltpu.sync_copy(acc.at[pl.ds(zbase + k * 2000, 2000)], m0_v)
            pltpu.sync_copy(m0_v, out_hbm.at[pl.ds(obase + k * 2000, 2000)])
        pltpu.sync_copy(acc.at[pl.ds(zbase + 6000, 256)],
                        m0_v.at[pl.ds(0, 256)])
        pltpu.sync_copy(m0_v.at[pl.ds(0, 256)],
                        out_hbm.at[pl.ds(obase + 6000, 256)])


def _tc1_body(deg, xr, du):
    d0 = deg[0:ROWS, :]
    d1 = deg[ROWS:2 * ROWS, :]
    dd = lax.rsqrt(d0 + d1 + 1.0)
    du[0:ROWS, :] = dd
    du[ROWS:2 * ROWS, :] = dd * xr[...]


def _tc2_body(accb, du, w1, b1, w2, v_out, vpk_out):
    a0 = accb[0:ROWS, :]
    a1 = accb[ROWS:2 * ROWS, :]
    d = du[0:ROWS, :]
    u = du[ROWS:2 * ROWS, :]
    y = d * (a0 + a1 + u)
    z0 = jnp.zeros_like(y)
    z1 = jnp.zeros_like(y)
    for j in range(16):
        h = jnp.maximum(y * w1[0, j] + b1[j], 0.0)
        z0 = z0 + h * w2[j, 0]
        z1 = z1 + h * w2[j, 1]
    v0 = d * z0
    v1 = d * z1
    v_out[0:ROWS, :] = v0
    v_out[ROWS:2 * ROWS, :] = v1
    hi = lax.bitcast_convert_type(
        v0.astype(jnp.bfloat16), jnp.uint16).astype(jnp.uint32) << 16
    lo = lax.bitcast_convert_type(
        v1.astype(jnp.bfloat16), jnp.uint16).astype(jnp.uint32)
    vpk_out[...] = lax.bitcast_convert_type(hi | lo, _i32)


def _tc3_body(accc, v, du, b2, o0, o1):
    a00 = accc[0:ROWS, :]
    a10 = accc[ROWS:2 * ROWS, :]
    a01 = accc[2 * ROWS:3 * ROWS, :]
    a11 = accc[3 * ROWS:4 * ROWS, :]
    v0 = v[0:ROWS, :]
    v1 = v[ROWS:2 * ROWS, :]
    d = du[0:ROWS, :]
    q0 = d * (a00 + a01 + v0) + b2[0]
    q1 = d * (a10 + a11 + v1) + b2[1]
    m = jnp.maximum(q0, q1)
    lz = m + jnp.log(jnp.exp(q0 - m) + jnp.exp(q1 - m))
    o0[...] = q0 - lz
    o1[...] = q1 - lz


_vspec = pl.BlockSpec(memory_space=pltpu.VMEM)
_smem_spec = pl.BlockSpec(memory_space=pltpu.SMEM)


def _sds(rows, dtype=_f32):
    return jax.ShapeDtypeStruct((rows, 128), dtype)


def kernel(x, edge_index, edge_weight, W1, b1, W2, b2):
    rowe = edge_index[0]
    cole = edge_index[1]
    we = edge_weight
    xp = jnp.pad(x[:, 0], (0, NPAD - N)).reshape(ROWS, 128)

    deg = _sc_degree(cole, we)

    du = pl.pallas_call(
        _tc1_body,
        in_specs=[_vspec, _vspec],
        out_specs=_vspec,
        out_shape=_sds(2 * ROWS),
    )(deg.reshape(2 * ROWS, 128), xp)

    accB = _sc_spmv1(rowe, cole, we, du.reshape(NC * NPAD))

    v, vpk = pl.pallas_call(
        _tc2_body,
        in_specs=[_vspec, _vspec, _smem_spec, _smem_spec, _smem_spec],
        out_specs=[_vspec, _vspec],
        out_shape=[_sds(2 * ROWS), _sds(ROWS, _i32)],
    )(accB.reshape(2 * ROWS, 128), du, W1, b1, W2)

    accC = _sc_spmv2(rowe, cole, we, vpk.reshape(NPAD))

    o0, o1 = pl.pallas_call(
        _tc3_body,
        in_specs=[_vspec, _vspec, _vspec, _smem_spec],
        out_specs=[_vspec, _vspec],
        out_shape=[_sds(ROWS), _sds(ROWS)],
    )(accC.reshape(4 * ROWS, 128), v, du, b2)

    return jnp.stack([o0.reshape(NPAD)[:N], o1.reshape(NPAD)[:N]], axis=1)


# speedup vs baseline: 1.0365x; 1.0093x over previous
import functools

import jax
import jax.numpy as jnp
from jax import lax
from jax.experimental import pallas as pl
from jax.experimental.pallas import tpu as pltpu
from jax.experimental.pallas import tpu_sc as plsc

N = 100000
E = 3200000
NPAD = 100096
ROWS = 782
SLICE = NPAD // 16
CHUNK = 2000
CHUNK_A = 25000
NC = 2
NS = 16
NW = NC * NS
EPW = E // NW
NT_AB = EPW // CHUNK
NT_A = EPW // CHUNK_A

_mesh = plsc.VectorSubcoreMesh(
    core_axis_name="c", subcore_axis_name="s", num_cores=NC, num_subcores=NS)
_sc_params = pltpu.CompilerParams(needs_layout_passes=False)

_f32 = jnp.float32
_i32 = jnp.int32
_out2 = jax.ShapeDtypeStruct((NC * NPAD,), _f32)
_out4 = jax.ShapeDtypeStruct((2 * NC * NPAD,), _f32)


def _fill_zero(bounce_v):
    z16 = jnp.zeros((16,), _f32)

    def zb(i, _):
        bounce_v[pl.ds(i * 16, 16)] = z16
        return _

    lax.fori_loop(0, SLICE // 16, zb, None)


def _zero_acc(bounce_v, acc_sh, sid):
    _fill_zero(bounce_v)
    pltpu.sync_copy(bounce_v, acc_sh.at[pl.ds(sid * SLICE, SLICE)])


def _write_out(acc_sh, bounce_v, out_hbm, obase, sid):
    pltpu.sync_copy(acc_sh.at[pl.ds(sid * SLICE, SLICE)], bounce_v)
    pltpu.sync_copy(bounce_v, out_hbm.at[pl.ds(obase + sid * SLICE, SLICE)])


@functools.partial(
    pl.kernel,
    out_type=_out2,
    mesh=_mesh,
    compiler_params=_sc_params,
    scratch_types=[
        pltpu.VMEM_SHARED((NPAD,), _f32),
        pltpu.VMEM((SLICE,), _f32),
        pltpu.VMEM((CHUNK_A,), _i32),
        pltpu.VMEM((CHUNK_A,), _i32),
        pltpu.VMEM((CHUNK_A,), _f32),
        pltpu.VMEM((CHUNK_A,), _f32),
        pltpu.SemaphoreType.DMA,
        pltpu.SemaphoreType.DMA,
        pltpu.SemaphoreType.DMA,
        pltpu.SemaphoreType.DMA,
    ],
)
def _sc_degree(col_hbm, w_hbm, out_hbm, acc_sh, bounce_v, col_v0, col_v1,
               w_v0, w_v1, sem0, sem1, ssem0, ssem1):
    cid = lax.axis_index("c")
    sid = lax.axis_index("s")
    wid = sid * NC + cid
    base = wid * EPW
    sems = (sem0, sem1)
    ssems = (ssem0, ssem1)
    col_b = (col_v0, col_v1)
    w_b = (w_v0, w_v1)

    def start(t, b):
        sl = pl.ds(base + t * CHUNK_A, CHUNK_A)
        pltpu.async_copy(col_hbm.at[sl], col_b[b], sems[b])
        pltpu.async_copy(w_hbm.at[sl], w_b[b], sems[b])

    def wait(t, b):
        sl = pl.ds(base + t * CHUNK_A, CHUNK_A)
        pltpu.make_async_copy(col_hbm.at[sl], col_b[b], sems[b]).wait()
        pltpu.make_async_copy(w_hbm.at[sl], w_b[b], sems[b]).wait()

    def scat_wait(b):
        pltpu.make_async_copy(
            w_b[b], acc_sh.at[col_b[b]], ssems[b]).wait()

    _zero_acc(bounce_v, acc_sh, sid)
    plsc.subcore_barrier()

    start(0, 0)

    def body(i, _):
        for b in (0, 1):
            t = 2 * i + b

            def free_and_prefetch():
                start(t + 1, 1 - b)

            if b == 0:
                pl.when(i > 0)(lambda: scat_wait(1))
                free_and_prefetch()
            else:
                scat_wait(0)
                pl.when(i < NT_A // 2 - 1)(free_and_prefetch)
            wait(t, b)
            pltpu.async_copy(w_b[b], acc_sh.at[col_b[b]], ssems[b], add=True)
        return _

    lax.fori_loop(0, NT_A // 2, body, None)
    scat_wait(1)
    plsc.subcore_barrier()
    _write_out(acc_sh, bounce_v, out_hbm, cid * NPAD, sid)


@functools.partial(
    pl.kernel,
    out_type=_out2,
    mesh=_mesh,
    compiler_params=_sc_params,
    scratch_types=[
        pltpu.VMEM_SHARED((NPAD,), _f32),
        pltpu.VMEM((SLICE,), _f32),
        pltpu.VMEM((NPAD,), _f32),
        pltpu.VMEM((CHUNK,), _i32),
        pltpu.VMEM((CHUNK,), _i32),
        pltpu.VMEM((CHUNK,), _i32),
        pltpu.VMEM((CHUNK,), _i32),
        pltpu.VMEM((CHUNK,), _f32),
        pltpu.VMEM((CHUNK,), _f32),
        pltpu.VMEM((CHUNK,), _f32),
        pltpu.VMEM((CHUNK,), _f32),
        pltpu.SemaphoreType.DMA,
        pltpu.SemaphoreType.DMA,
        pltpu.SemaphoreType.DMA,
        pltpu.SemaphoreType.DMA,
    ],
)
def _sc_spmv1(row_hbm, col_hbm, w_hbm, du_hbm, out_hbm,
              acc_sh, bounce_v, tab_v, row_v0, row_v1, col_v0, col_v1,
              w_v0, w_v1, m_v0, m_v1, sem0, sem1, ssem0, ssem1):
    cid = lax.axis_index("c")
    sid = lax.axis_index("s")
    wid = sid * NC + cid
    base = wid * EPW
    sems = (sem0, sem1)
    ssems = (ssem0, ssem1)
    row_b = (row_v0, row_v1)
    col_b = (col_v0, col_v1)
    w_b = (w_v0, w_v1)
    m_b = (m_v0, m_v1)

    def start(t, b):
        sl = pl.ds(base + t * CHUNK, CHUNK)
        pltpu.async_copy(row_hbm.at[sl], row_b[b], sems[b])
        pltpu.async_copy(col_hbm.at[sl], col_b[b], sems[b])
        pltpu.async_copy(w_hbm.at[sl], w_b[b], sems[b])

    def wait(t, b):
        sl = pl.ds(base + t * CHUNK, CHUNK)
        pltpu.make_async_copy(row_hbm.at[sl], row_b[b], sems[b]).wait()
        pltpu.make_async_copy(col_hbm.at[sl], col_b[b], sems[b]).wait()
        pltpu.make_async_copy(w_hbm.at[sl], w_b[b], sems[b]).wait()

    def scat_wait(b):
        pltpu.make_async_copy(
            m_b[b], acc_sh.at[col_b[b]], ssems[b]).wait()

    def gather_mul(b):
        rv, wv, mv = row_b[b], w_b[b], m_b[b]

        @plsc.parallel_loop(0, CHUNK, 16, unroll=8)
        def gbody(i):
            sl = pl.ds(i, 16)
            t16 = plsc.load_gather(tab_v, [rv[sl]])
            mv[sl] = wv[sl] * t16

    start(0, 0)
    _zero_acc(bounce_v, acc_sh, sid)
    pltpu.sync_copy(du_hbm.at[pl.ds(NPAD, NPAD)], tab_v)
    plsc.subcore_barrier()

    def body(i, _):
        for b in (0, 1):
            t = 2 * i + b

            def free_and_prefetch():
                start(t + 1, 1 - b)

            if b == 0:
                pl.when(i > 0)(lambda: scat_wait(1))
                free_and_prefetch()
            else:
                scat_wait(0)
                pl.when(i < NT_AB // 2 - 1)(free_and_prefetch)
            wait(t, b)
            gather_mul(b)
            pltpu.async_copy(m_b[b], acc_sh.at[col_b[b]], ssems[b], add=True)
        return _

    lax.fori_loop(0, NT_AB // 2, body, None)
    scat_wait(1)
    plsc.subcore_barrier()
    _write_out(acc_sh, bounce_v, out_hbm, cid * NPAD, sid)


@functools.partial(
    pl.kernel,
    out_type=_out4,
    mesh=_mesh,
    compiler_params=_sc_params,
    scratch_types=[
        pltpu.VMEM_SHARED((NPAD,), _f32),
        pltpu.VMEM_SHARED((NPAD,), _f32),
        pltpu.VMEM((NPAD,), _i32),
        pltpu.VMEM((CHUNK,), _i32),
        pltpu.VMEM((CHUNK,), _i32),
        pltpu.VMEM((CHUNK,), _i32),
        pltpu.VMEM((CHUNK,), _i32),
        pltpu.VMEM((CHUNK,), _f32),
        pltpu.VMEM((CHUNK,), _f32),
        pltpu.VMEM((CHUNK,), _f32),
        pltpu.VMEM((CHUNK,), _f32),
        pltpu.SemaphoreType.DMA,
        pltpu.SemaphoreType.DMA,
        pltpu.SemaphoreType.DMA,
    ],
)
def _sc_spmv2(row_hbm, col_hbm, w_hbm, vpk_hbm, out_hbm,
              acc0_sh, acc1_sh, tab_v, row_v0, row_v1,
              col_v0, col_v1, w_v0, w_v1, m0_v, m1_v,
              sem0, sem1, ssem):
    cid = lax.axis_index("c")
    sid = lax.axis_index("s")
    wid = sid * NC + cid
    base = wid * EPW
    sems = (sem0, sem1)
    row_b = (row_v0, row_v1)
    col_b = (col_v0, col_v1)
    w_b = (w_v0, w_v1)

    def start(t, b):
        sl = pl.ds(base + t * CHUNK, CHUNK)
        pltpu.async_copy(row_hbm.at[sl], row_b[b], sems[b])
        pltpu.async_copy(col_hbm.at[sl], col_b[b], sems[b])
        pltpu.async_copy(w_hbm.at[sl], w_b[b], sems[b])

    def wait(t, b):
        sl = pl.ds(base + t * CHUNK, CHUNK)
        pltpu.make_async_copy(row_hbm.at[sl], row_b[b], sems[b]).wait()
        pltpu.make_async_copy(col_hbm.at[sl], col_b[b], sems[b]).wait()
        pltpu.make_async_copy(w_hbm.at[sl], w_b[b], sems[b]).wait()

    hi_mask = jnp.full((16,), -65536, _i32)

    def gather_mul(b):
        rv, wv, m0v, m1v = row_b[b], w_b[b], m0_v, m1_v

        @plsc.parallel_loop(0, CHUNK, 16, unroll=8)
        def gbody(i):
            sl = pl.ds(i, 16)
            pk = plsc.load_gather(tab_v, [rv[sl]])
            v0 = plsc.bitcast(pk & hi_mask, _f32)
            v1 = plsc.bitcast(pk << 16, _f32)
            w16 = wv[sl]
            m0v[sl] = w16 * v0
            m1v[sl] = w16 * v1

    start(0, 0)
    z16 = jnp.zeros((16,), _f32)

    def zb(i, _):
        m0_v[pl.ds(i * 16, 16)] = z16
        return _

    lax.fori_loop(0, CHUNK // 16, zb, None)
    zbase = sid * SLICE
    for acc in (acc0_sh, acc1_sh):
        for k in range(3):
            pltpu.sync_copy(m0_v, acc.at[pl.ds(zbase + k * 2000, 2000)])
        pltpu.sync_copy(m0_v.at[pl.ds(0, 256)],
                        acc.at[pl.ds(zbase + 6000, 256)])
    pltpu.sync_copy(vpk_hbm, tab_v)
    plsc.subcore_barrier()

    def scat0_wait(b):
        pltpu.make_async_copy(m0_v, acc0_sh.at[col_b[b]], ssem).wait()

    def body(i, _):
        for b in (0, 1):
            t = 2 * i + b

            def prefetch():
                start(t + 1, 1 - b)

            if b == 0:
                pl.when(i > 0)(lambda: scat0_wait(1))
                prefetch()
            else:
                scat0_wait(0)
                pl.when(i < NT_AB // 2 - 1)(prefetch)
            wait(t, b)
            gather_mul(b)
            pltpu.async_copy(m0_v, acc0_sh.at[col_b[b]], ssem, add=True)
            pltpu.sync_copy(m1_v, acc1_sh.at[col_b[b]], add=True)
        return _

    lax.fori_loop(0, NT_AB // 2, body, None)
    scat0_wait(1)
    plsc.subcore_barrier()
    for f, acc in ((0, acc0_sh), (1, acc1_sh)):
        obase = cid * 2 * NPAD + f * NPAD + sid * SLICE
        for k in range(3):
            pltpu.sync_copy(acc.at[pl.ds(zbase + k * 2000, 2000)], m0_v)
            pltpu.sync_copy(m0_v, out_hbm.at[pl.ds(obase + k * 2000, 2000)])
        pltpu.sync_copy(acc.at[pl.ds(zbase + 6000, 256)],
                        m0_v.at[pl.ds(0, 256)])
        pltpu.sync_copy(m0_v.at[pl.ds(0, 256)],
                        out_hbm.at[pl.ds(obase + 6000, 256)])


def _tc1_body(deg, xr, du):
    d0 = deg[0:ROWS, :]
    d1 = deg[ROWS:2 * ROWS, :]
    dd = lax.rsqrt(d0 + d1 + 1.0)
    du[0:ROWS, :] = dd
    du[ROWS:2 * ROWS, :] = dd * xr[...]


def _tc2_body(accb, du, w1, b1, w2, v_out, vpk_out):
    a0 = accb[0:ROWS, :]
    a1 = accb[ROWS:2 * ROWS, :]
    d = du[0:ROWS, :]
    u = du[ROWS:2 * ROWS, :]
    y = d * (a0 + a1 + u)
    z0 = jnp.zeros_like(y)
    z1 = jnp.zeros_like(y)
    for j in range(16):
        h = jnp.maximum(y * w1[0, j] + b1[j], 0.0)
        z0 = z0 + h * w2[j, 0]
        z1 = z1 + h * w2[j, 1]
    v0 = d * z0
    v1 = d * z1
    v_out[0:ROWS, :] = v0
    v_out[ROWS:2 * ROWS, :] = v1
    hi = lax.bitcast_convert_type(
        v0.astype(jnp.bfloat16), jnp.uint16).astype(jnp.uint32) << 16
    lo = lax.bitcast_convert_type(
        v1.astype(jnp.bfloat16), jnp.uint16).astype(jnp.uint32)
    vpk_out[...] = lax.bitcast_convert_type(hi | lo, _i32)


def _tc3_body(accc, v, du, b2, o0, o1):
    a00 = accc[0:ROWS, :]
    a10 = accc[ROWS:2 * ROWS, :]
    a01 = accc[2 * ROWS:3 * ROWS, :]
    a11 = accc[3 * ROWS:4 * ROWS, :]
    v0 = v[0:ROWS, :]
    v1 = v[ROWS:2 * ROWS, :]
    d = du[0:ROWS, :]
    q0 = d * (a00 + a01 + v0) + b2[0]
    q1 = d * (a10 + a11 + v1) + b2[1]
    m = jnp.maximum(q0, q1)
    lz = m + jnp.log(jnp.exp(q0 - m) + jnp.exp(q1 - m))
    o0[...] = q0 - lz
    o1[...] = q1 - lz


_vspec = pl.BlockSpec(memory_space=pltpu.VMEM)
_smem_spec = pl.BlockSpec(memory_space=pltpu.SMEM)


def _sds(rows, dtype=_f32):
    return jax.ShapeDtypeStruct((rows, 128), dtype)


def kernel(x, edge_index, edge_weight, W1, b1, W2, b2):
    rowe = edge_index[0]
    cole = edge_index[1]
    we = edge_weight
    xp = jnp.pad(x[:, 0], (0, NPAD - N)).reshape(ROWS, 128)

    deg = _sc_degree(cole, we)

    du = pl.pallas_call(
        _tc1_body,
        in_specs=[_vspec, _vspec],
        out_specs=_vspec,
        out_shape=_sds(2 * ROWS),
    )(deg.reshape(2 * ROWS, 128), xp)

    accB = _sc_spmv1(rowe, cole, we, du.reshape(NC * NPAD))

    v, vpk = pl.pallas_call(
        _tc2_body,
        in_specs=[_vspec, _vspec, _smem_spec, _smem_spec, _smem_spec],
        out_specs=[_vspec, _vspec],
        out_shape=[_sds(2 * ROWS), _sds(ROWS, _i32)],
    )(accB.reshape(2 * ROWS, 128), du, W1, b1, W2)

    accC = _sc_spmv2(rowe, cole, we, vpk.reshape(NPAD))

    o0, o1 = pl.pallas_call(
        _tc3_body,
        in_specs=[_vspec, _vspec, _vspec, _smem_spec],
        out_specs=[_vspec, _vspec],
        out_shape=[_sds(ROWS), _sds(ROWS)],
    )(accC.reshape(4 * ROWS, 128), v, du, b2)

    return jnp.stack([o0.reshape(NPAD)[:N], o1.reshape(NPAD)[:N]], axis=1)
